# Initial kernel scaffold; baseline (speedup 1.0000x reference)
import jax
import jax.numpy as jnp
from jax.experimental import pallas as pl


def kernel(input_feat, edge_index, pseudo, Wg1, mu1, sigma1, Wroot1, b1, Wg2, mu2, sigma2, Wroot2, b2):
    raise NotImplementedError("write your pallas kernel here")


# speedup vs baseline: 1.1145x; 1.1145x over previous
import functools

import jax
import jax.numpy as jnp
from jax import lax
from jax.experimental import pallas as pl
from jax.experimental.pallas import tpu as pltpu
from jax.experimental.pallas import tpu_sc as plsc

N = 40962
E = 245772
D = 128
K = 10
EPS = 1e-15

NPAD = 41472
EPAD = 246784
WG_E = 7712
TG = 32
NGS = WG_E // TG
WS_E = 15424
TS = 64
NSS = WS_E // TS
RT = 6912
RTT = 7040
ZB_R = 88
ZSTEPS = 5
WB_R = 144
WSTEPS = 3
BE = 512
BN = 256

_MESH = plsc.VectorSubcoreMesh(core_axis_name="c", subcore_axis_name="s")


@functools.partial(
    pl.kernel,
    out_type=jax.ShapeDtypeStruct((EPAD, D), jnp.float32),
    mesh=_MESH,
    scratch_types=[
        pltpu.VMEM((TG,), jnp.int32),
        pltpu.VMEM((TG, D), jnp.float32),
        pltpu.SemaphoreType.DMA,
    ],
)
def _gather_k(x_hbm, src_hbm, gx_hbm, idx_v, rows_v, sem):
    c = lax.axis_index("c")
    s = lax.axis_index("s")
    base = (s * 2 + c) * WG_E

    def step(i, carry):
        e0 = base + i * TG
        pltpu.sync_copy(src_hbm.at[pl.ds(e0, TG)], idx_v)
        pltpu.async_copy(x_hbm.at[idx_v], rows_v, sem).wait()
        pltpu.sync_copy(rows_v, gx_hbm.at[pl.ds(e0, TG)])
        return carry

    lax.fori_loop(0, NGS, step, 0)


def _clamp_idx(idx_v, idx2_v, base):
    for u in range(TS // 16):
        v = idx_v[pl.ds(u * 16, 16)]
        local = v - base
        inb = (local >= 0) & (local < RT)
        idx2_v[pl.ds(u * 16, 16)] = jnp.where(inb, local, RT)


@functools.partial(
    pl.kernel,
    out_type=jax.ShapeDtypeStruct((NPAD, D), jnp.float32),
    mesh=_MESH,
    scratch_types=[
        pltpu.VMEM_SHARED((RTT, D), jnp.float32),
        pltpu.VMEM((TS,), jnp.int32),
        pltpu.VMEM((TS,), jnp.int32),
        pltpu.VMEM((TS, D), jnp.float32),
        pltpu.VMEM((ZB_R, D), jnp.float32),
        pltpu.VMEM((WB_R, D), jnp.float32),
    ],
)
def _scatter_k(msg_hbm, dst_hbm, zro_hbm, agg_hbm, table, idx_v, idx2_v,
               rows_v, zb, wb):
    c = lax.axis_index("c")
    s = lax.axis_index("s")

    pltpu.sync_copy(zro_hbm, zb)
    for j in (0, 1, 2):
        cid = 3 * c + j
        base = cid * RT

        def zstep(t, carry):
            pltpu.sync_copy(zb, table.at[pl.ds(s * 440 + t * ZB_R, ZB_R)])
            return carry

        lax.fori_loop(0, ZSTEPS, zstep, 0)
        plsc.subcore_barrier()

        def step(i, carry):
            e0 = s * WS_E + i * TS
            pltpu.sync_copy(dst_hbm.at[pl.ds(e0, TS)], idx_v)
            pltpu.sync_copy(msg_hbm.at[pl.ds(e0, TS)], rows_v)
            _clamp_idx(idx_v, idx2_v, base)
            pltpu.sync_copy(rows_v, table.at[idx2_v], add=True)
            return carry

        lax.fori_loop(0, NSS, step, 0)
        plsc.subcore_barrier()

        def wstep(t, carry):
            r0 = s * 432 + t * WB_R
            pltpu.sync_copy(table.at[pl.ds(r0, WB_R)], wb)
            pltpu.sync_copy(wb, agg_hbm.at[pl.ds(base + r0, WB_R)])
            return carry

        lax.fori_loop(0, WSTEPS, wstep, 0)
        plsc.subcore_barrier()


@functools.partial(
    pl.kernel,
    out_type=jax.ShapeDtypeStruct((NPAD, D), jnp.float32),
    mesh=_MESH,
    scratch_types=[
        pltpu.VMEM_SHARED((RTT, D), jnp.float32),
        pltpu.VMEM((TS,), jnp.int32),
        pltpu.VMEM((TS,), jnp.int32),
        pltpu.VMEM((TS, D), jnp.float32),
        pltpu.VMEM((ZB_R, D), jnp.float32),
        pltpu.VMEM((WB_R, D), jnp.float32),
    ],
)
def _cnt_k(dst_hbm, zro_hbm, cnt_hbm, table, idx_v, idx2_v, ones_v, zb, wb):
    c = lax.axis_index("c")
    s = lax.axis_index("s")

    pltpu.sync_copy(zro_hbm, zb)
    ones16 = jnp.ones((16,), jnp.float32)
    for i in range(TS):
        for u in range(D // 16):
            ones_v[i, pl.ds(u * 16, 16)] = ones16
    for j in (0, 1, 2):
        cid = 3 * c + j
        base = cid * RT

        def zstep(t, carry):
            pltpu.sync_copy(zb, table.at[pl.ds(s * 440 + t * ZB_R, ZB_R)])
            return carry

        lax.fori_loop(0, ZSTEPS, zstep, 0)
        plsc.subcore_barrier()

        def step(i, carry):
            e0 = s * WS_E + i * TS
            pltpu.sync_copy(dst_hbm.at[pl.ds(e0, TS)], idx_v)
            _clamp_idx(idx_v, idx2_v, base)
            pltpu.sync_copy(ones_v, table.at[idx2_v], add=True)
            return carry

        lax.fori_loop(0, NSS, step, 0)
        plsc.subcore_barrier()

        def wstep(t, carry):
            r0 = s * 432 + t * WB_R
            pltpu.sync_copy(table.at[pl.ds(r0, WB_R)], wb)
            pltpu.sync_copy(wb, cnt_hbm.at[pl.ds(base + r0, WB_R)])
            return carry

        lax.fori_loop(0, WSTEPS, wstep, 0)
        plsc.subcore_barrier()


def _combine_body(gx_ref, ps_ref, wg_ref, c2_ref, c1_ref, c0_ref, sel_ref,
                  out_ref):
    ps = ps_ref[...]
    g = jnp.exp(
        jnp.dot(ps * ps, c2_ref[...], preferred_element_type=jnp.float32)
        + jnp.dot(ps, c1_ref[...], preferred_element_type=jnp.float32)
        + c0_ref[...]
    )
    gb = jnp.dot(g[:, :16].astype(jnp.bfloat16), sel_ref[...],
                 preferred_element_type=jnp.float32)
    gx = gx_ref[...].astype(jnp.bfloat16)
    acc = jnp.zeros((BE, D), jnp.float32)
    for k in range(K):
        t = jnp.dot(gx, wg_ref[:, k * D:(k + 1) * D],
                    preferred_element_type=jnp.float32)
        acc = acc + t * gb[:, k * D:(k + 1) * D]
    out_ref[...] = acc


def _combine(gx, ps_p, wg, c2, c1, c0, sel):
    return pl.pallas_call(
        _combine_body,
        grid=(EPAD // BE,),
        in_specs=[
            pl.BlockSpec((BE, D), lambda i: (i, 0)),
            pl.BlockSpec((BE, 8), lambda i: (i, 0)),
            pl.BlockSpec((D, D * K), lambda i: (0, 0)),
            pl.BlockSpec((8, 128), lambda i: (0, 0)),
            pl.BlockSpec((8, 128), lambda i: (0, 0)),
            pl.BlockSpec((1, 128), lambda i: (0, 0)),
            pl.BlockSpec((16, D * K), lambda i: (0, 0)),
        ],
        out_specs=pl.BlockSpec((BE, D), lambda i: (i, 0)),
        out_shape=jax.ShapeDtypeStruct((EPAD, D), jnp.float32),
    )(gx, ps_p, wg, c2, c1, c0, sel)


def _post_body_res(agg_ref, cnt_ref, x_ref, wroot_ref, b_ref, res_ref, out_ref):
    _post_common(agg_ref, cnt_ref, x_ref, wroot_ref, b_ref, res_ref, out_ref)


def _post_body_nores(agg_ref, cnt_ref, x_ref, wroot_ref, b_ref, out_ref):
    _post_common(agg_ref, cnt_ref, x_ref, wroot_ref, b_ref, None, out_ref)


def _post_common(agg_ref, cnt_ref, x_ref, wroot_ref, b_ref, res_ref, out_ref):
    cnt = jnp.maximum(cnt_ref[:, 0:1], 1.0)
    o = agg_ref[...] / cnt
    o = o + jnp.dot(x_ref[...], wroot_ref[...], preferred_element_type=jnp.float32)
    o = o + b_ref[...]
    if res_ref is not None:
        o = o + res_ref[...]
    out_ref[...] = jnp.maximum(o, 0.0)


def _post(agg, cnt, x_p, wroot, brow, res_p):
    specs = [
        pl.BlockSpec((BN, D), lambda i: (i, 0)),
        pl.BlockSpec((BN, D), lambda i: (i, 0)),
        pl.BlockSpec((BN, D), lambda i: (i, 0)),
        pl.BlockSpec((D, D), lambda i: (0, 0)),
        pl.BlockSpec((1, D), lambda i: (0, 0)),
    ]
    args = [agg, cnt, x_p, wroot, brow]
    body = _post_body_nores
    if res_p is not None:
        specs.append(pl.BlockSpec((BN, D), lambda i: (i, 0)))
        args.append(res_p)
        body = _post_body_res
    return pl.pallas_call(
        body,
        grid=(NPAD // BN,),
        in_specs=specs,
        out_specs=pl.BlockSpec((BN, D), lambda i: (i, 0)),
        out_shape=jax.ShapeDtypeStruct((NPAD, D), jnp.float32),
    )(*args)


def _gauss_coeffs(mu, sigma):
    iv = 1.0 / (sigma.astype(jnp.float32) ** 2 + EPS)
    c2 = jnp.zeros((8, 128), jnp.float32)
    c2 = c2.at[0, :K].set(-0.5 * iv[:, 0]).at[1, :K].set(-0.5 * iv[:, 1])
    c1 = jnp.zeros((8, 128), jnp.float32)
    c1 = c1.at[0, :K].set(iv[:, 0] * mu[:, 0]).at[1, :K].set(iv[:, 1] * mu[:, 1])
    c0v = -0.5 * (iv[:, 0] * mu[:, 0] ** 2 + iv[:, 1] * mu[:, 1] ** 2)
    c0 = jnp.full((1, 128), -1e30, jnp.float32).at[0, :K].set(c0v)
    return c2, c1, c0


def kernel(input_feat, edge_index, pseudo, Wg1, mu1, sigma1, Wroot1, b1,
           Wg2, mu2, sigma2, Wroot2, b2):
    src = edge_index[0]
    dst = edge_index[1]
    ep = EPAD - E
    src_p = jnp.concatenate([src, jnp.zeros((ep,), jnp.int32)])
    dst_p = jnp.concatenate([dst, jnp.full((ep,), N, jnp.int32)])
    ps_p = jnp.zeros((EPAD, 8), jnp.float32).at[:E, :2].set(pseudo)
    x_p = jnp.zeros((NPAD, D), jnp.float32).at[:N].set(input_feat)
    zro = jnp.zeros((ZB_R, D), jnp.float32)
    c2a, c1a, c0a = _gauss_coeffs(mu1, sigma1)
    c2b, c1b, c0b = _gauss_coeffs(mu2, sigma2)
    sel = jnp.zeros((16, D * K), jnp.bfloat16)
    for k in range(K):
        sel = sel.at[k, k * D:(k + 1) * D].set(1)
    wg1b = Wg1.astype(jnp.bfloat16)
    wg2b = Wg2.astype(jnp.bfloat16)

    cnt = _cnt_k(dst_p, zro)

    gx1 = _gather_k(x_p, src_p)
    msg1 = _combine(gx1, ps_p, wg1b, c2a, c1a, c0a, sel)
    agg1 = _scatter_k(msg1, dst_p, zro)
    h = _post(agg1, cnt, x_p, Wroot1, b1.reshape(1, D), None)

    gx2 = _gather_k(h, src_p)
    msg2 = _combine(gx2, ps_p, wg2b, c2b, c1b, c0b, sel)
    agg2 = _scatter_k(msg2, dst_p, zro)
    out = _post(agg2, cnt, h, Wroot2, b2.reshape(1, D), x_p)
    return out[:N]


# speedup vs baseline: 1.2786x; 1.1473x over previous
import functools

import jax
import jax.numpy as jnp
from jax import lax
from jax.experimental import pallas as pl
from jax.experimental.pallas import tpu as pltpu
from jax.experimental.pallas import tpu_sc as plsc

N = 40962
E = 245772
D = 128
K = 10
EPS = 1e-15

NPAD = 41472
EPAD = 249856
WG_E = 7808
TG = 128
NGS = WG_E // TG
WS_E = 15616
TS = 128
NSS = WS_E // TS
RT = 6912
RTT = 7040
ZB_R = 88
ZSTEPS = 5
WB_R = 144
WSTEPS = 3
BE = 512
BN = 256

_MESH = plsc.VectorSubcoreMesh(core_axis_name="c", subcore_axis_name="s")


@functools.partial(
    pl.kernel,
    out_type=jax.ShapeDtypeStruct((EPAD, D), jnp.float32),
    mesh=_MESH,
    scratch_types=[
        pltpu.VMEM((TG,), jnp.int32),
        pltpu.VMEM((TG, D), jnp.float32),
        pltpu.SemaphoreType.DMA,
    ],
)
def _gather_k(x_hbm, src_hbm, gx_hbm, idx_v, rows_v, sem):
    c = lax.axis_index("c")
    s = lax.axis_index("s")
    base = (s * 2 + c) * WG_E

    def step(i, carry):
        e0 = base + i * TG
        pltpu.sync_copy(src_hbm.at[pl.ds(e0, TG)], idx_v)
        pltpu.async_copy(x_hbm.at[idx_v], rows_v, sem).wait()
        pltpu.sync_copy(rows_v, gx_hbm.at[pl.ds(e0, TG)])
        return carry

    lax.fori_loop(0, NGS, step, 0)


def _clamp_idx(idx_v, idx2_v, base):
    for u in range(TS // 16):
        v = idx_v[pl.ds(u * 16, 16)]
        local = v - base
        inb = (local >= 0) & (local < RT)
        idx2_v[pl.ds(u * 16, 16)] = jnp.where(inb, local, RT)


@functools.partial(
    pl.kernel,
    out_type=jax.ShapeDtypeStruct((NPAD, D), jnp.float32),
    mesh=_MESH,
    scratch_types=[
        pltpu.VMEM_SHARED((RTT, D), jnp.float32),
        pltpu.VMEM((TS,), jnp.int32),
        pltpu.VMEM((TS,), jnp.int32),
        pltpu.VMEM((TS, D), jnp.float32),
        pltpu.VMEM((ZB_R, D), jnp.float32),
        pltpu.VMEM((WB_R, D), jnp.float32),
    ],
)
def _scatter_k(msg_hbm, dst_hbm, zro_hbm, agg_hbm, table, idx_v, idx2_v,
               rows_v, zb, wb):
    c = lax.axis_index("c")
    s = lax.axis_index("s")

    pltpu.sync_copy(zro_hbm, zb)
    for j in (0, 1, 2):
        cid = 3 * c + j
        base = cid * RT

        def zstep(t, carry):
            pltpu.sync_copy(zb, table.at[pl.ds(s * 440 + t * ZB_R, ZB_R)])
            return carry

        lax.fori_loop(0, ZSTEPS, zstep, 0)
        plsc.subcore_barrier()

        def step(i, carry):
            e0 = s * WS_E + i * TS
            pltpu.sync_copy(dst_hbm.at[pl.ds(e0, TS)], idx_v)
            pltpu.sync_copy(msg_hbm.at[pl.ds(e0, TS)], rows_v)
            _clamp_idx(idx_v, idx2_v, base)
            pltpu.sync_copy(rows_v, table.at[idx2_v], add=True)
            return carry

        lax.fori_loop(0, NSS, step, 0)
        plsc.subcore_barrier()

        def wstep(t, carry):
            r0 = s * 432 + t * WB_R
            pltpu.sync_copy(table.at[pl.ds(r0, WB_R)], wb)
            pltpu.sync_copy(wb, agg_hbm.at[pl.ds(base + r0, WB_R)])
            return carry

        lax.fori_loop(0, WSTEPS, wstep, 0)
        plsc.subcore_barrier()


@functools.partial(
    pl.kernel,
    out_type=jax.ShapeDtypeStruct((NPAD, D), jnp.float32),
    mesh=_MESH,
    scratch_types=[
        pltpu.VMEM_SHARED((RTT, D), jnp.float32),
        pltpu.VMEM((TS,), jnp.int32),
        pltpu.VMEM((TS,), jnp.int32),
        pltpu.VMEM((TS, D), jnp.float32),
        pltpu.VMEM((ZB_R, D), jnp.float32),
        pltpu.VMEM((WB_R, D), jnp.float32),
    ],
)
def _cnt_k(dst_hbm, zro_hbm, cnt_hbm, table, idx_v, idx2_v, ones_v, zb, wb):
    c = lax.axis_index("c")
    s = lax.axis_index("s")

    pltpu.sync_copy(zro_hbm, zb)
    ones16 = jnp.ones((16,), jnp.float32)
    for i in range(TS):
        for u in range(D // 16):
            ones_v[i, pl.ds(u * 16, 16)] = ones16
    for j in (0, 1, 2):
        cid = 3 * c + j
        base = cid * RT

        def zstep(t, carry):
            pltpu.sync_copy(zb, table.at[pl.ds(s * 440 + t * ZB_R, ZB_R)])
            return carry

        lax.fori_loop(0, ZSTEPS, zstep, 0)
        plsc.subcore_barrier()

        def step(i, carry):
            e0 = s * WS_E + i * TS
            pltpu.sync_copy(dst_hbm.at[pl.ds(e0, TS)], idx_v)
            _clamp_idx(idx_v, idx2_v, base)
            pltpu.sync_copy(ones_v, table.at[idx2_v], add=True)
            return carry

        lax.fori_loop(0, NSS, step, 0)
        plsc.subcore_barrier()

        def wstep(t, carry):
            r0 = s * 432 + t * WB_R
            pltpu.sync_copy(table.at[pl.ds(r0, WB_R)], wb)
            pltpu.sync_copy(wb, cnt_hbm.at[pl.ds(base + r0, WB_R)])
            return carry

        lax.fori_loop(0, WSTEPS, wstep, 0)
        plsc.subcore_barrier()


def _combine_body(gx_ref, ps_ref, wg_ref, c2_ref, c1_ref, c0_ref, sel_ref,
                  out_ref):
    ps = ps_ref[...]
    g = jnp.exp(
        jnp.dot(ps * ps, c2_ref[...], preferred_element_type=jnp.float32)
        + jnp.dot(ps, c1_ref[...], preferred_element_type=jnp.float32)
        + c0_ref[...]
    )
    gb = jnp.dot(g[:, :16].astype(jnp.bfloat16), sel_ref[...],
                 preferred_element_type=jnp.float32)
    gx = gx_ref[...].astype(jnp.bfloat16)
    acc = jnp.zeros((BE, D), jnp.float32)
    for k in range(K):
        t = jnp.dot(gx, wg_ref[:, k * D:(k + 1) * D],
                    preferred_element_type=jnp.float32)
        acc = acc + t * gb[:, k * D:(k + 1) * D]
    out_ref[...] = acc


def _combine(gx, ps_p, wg, c2, c1, c0, sel):
    return pl.pallas_call(
        _combine_body,
        grid=(EPAD // BE,),
        in_specs=[
            pl.BlockSpec((BE, D), lambda i: (i, 0)),
            pl.BlockSpec((BE, 8), lambda i: (i, 0)),
            pl.BlockSpec((D, D * K), lambda i: (0, 0)),
            pl.BlockSpec((8, 128), lambda i: (0, 0)),
            pl.BlockSpec((8, 128), lambda i: (0, 0)),
            pl.BlockSpec((1, 128), lambda i: (0, 0)),
            pl.BlockSpec((16, D * K), lambda i: (0, 0)),
        ],
        out_specs=pl.BlockSpec((BE, D), lambda i: (i, 0)),
        out_shape=jax.ShapeDtypeStruct((EPAD, D), jnp.float32),
    )(gx, ps_p, wg, c2, c1, c0, sel)


def _post_body_res(agg_ref, cnt_ref, x_ref, wroot_ref, b_ref, res_ref, out_ref):
    _post_common(agg_ref, cnt_ref, x_ref, wroot_ref, b_ref, res_ref, out_ref)


def _post_body_nores(agg_ref, cnt_ref, x_ref, wroot_ref, b_ref, out_ref):
    _post_common(agg_ref, cnt_ref, x_ref, wroot_ref, b_ref, None, out_ref)


def _post_common(agg_ref, cnt_ref, x_ref, wroot_ref, b_ref, res_ref, out_ref):
    cnt = jnp.maximum(cnt_ref[:, 0:1], 1.0)
    o = agg_ref[...] / cnt
    o = o + jnp.dot(x_ref[...], wroot_ref[...], preferred_element_type=jnp.float32)
    o = o + b_ref[...]
    if res_ref is not None:
        o = o + res_ref[...]
    out_ref[...] = jnp.maximum(o, 0.0)


def _post(agg, cnt, x_p, wroot, brow, res_p):
    specs = [
        pl.BlockSpec((BN, D), lambda i: (i, 0)),
        pl.BlockSpec((BN, D), lambda i: (i, 0)),
        pl.BlockSpec((BN, D), lambda i: (i, 0)),
        pl.BlockSpec((D, D), lambda i: (0, 0)),
        pl.BlockSpec((1, D), lambda i: (0, 0)),
    ]
    args = [agg, cnt, x_p, wroot, brow]
    body = _post_body_nores
    if res_p is not None:
        specs.append(pl.BlockSpec((BN, D), lambda i: (i, 0)))
        args.append(res_p)
        body = _post_body_res
    return pl.pallas_call(
        body,
        grid=(NPAD // BN,),
        in_specs=specs,
        out_specs=pl.BlockSpec((BN, D), lambda i: (i, 0)),
        out_shape=jax.ShapeDtypeStruct((NPAD, D), jnp.float32),
    )(*args)


def _gauss_coeffs(mu, sigma):
    iv = 1.0 / (sigma.astype(jnp.float32) ** 2 + EPS)
    c2 = jnp.zeros((8, 128), jnp.float32)
    c2 = c2.at[0, :K].set(-0.5 * iv[:, 0]).at[1, :K].set(-0.5 * iv[:, 1])
    c1 = jnp.zeros((8, 128), jnp.float32)
    c1 = c1.at[0, :K].set(iv[:, 0] * mu[:, 0]).at[1, :K].set(iv[:, 1] * mu[:, 1])
    c0v = -0.5 * (iv[:, 0] * mu[:, 0] ** 2 + iv[:, 1] * mu[:, 1] ** 2)
    c0 = jnp.full((1, 128), -1e30, jnp.float32).at[0, :K].set(c0v)
    return c2, c1, c0


def kernel(input_feat, edge_index, pseudo, Wg1, mu1, sigma1, Wroot1, b1,
           Wg2, mu2, sigma2, Wroot2, b2):
    src = edge_index[0]
    dst = edge_index[1]
    ep = EPAD - E
    src_p = jnp.concatenate([src, jnp.zeros((ep,), jnp.int32)])
    dst_p = jnp.concatenate([dst, jnp.full((ep,), N, jnp.int32)])
    ps_p = jnp.zeros((EPAD, 8), jnp.float32).at[:E, :2].set(pseudo)
    x_p = jnp.zeros((NPAD, D), jnp.float32).at[:N].set(input_feat)
    zro = jnp.zeros((ZB_R, D), jnp.float32)
    c2a, c1a, c0a = _gauss_coeffs(mu1, sigma1)
    c2b, c1b, c0b = _gauss_coeffs(mu2, sigma2)
    sel = jnp.zeros((16, D * K), jnp.bfloat16)
    for k in range(K):
        sel = sel.at[k, k * D:(k + 1) * D].set(1)
    wg1b = Wg1.astype(jnp.bfloat16)
    wg2b = Wg2.astype(jnp.bfloat16)

    cnt = _cnt_k(dst_p, zro)

    gx1 = _gather_k(x_p, src_p)
    msg1 = _combine(gx1, ps_p, wg1b, c2a, c1a, c0a, sel)
    agg1 = _scatter_k(msg1, dst_p, zro)
    h = _post(agg1, cnt, x_p, Wroot1, b1.reshape(1, D), None)

    gx2 = _gather_k(h, src_p)
    msg2 = _combine(gx2, ps_p, wg2b, c2b, c1b, c0b, sel)
    agg2 = _scatter_k(msg2, dst_p, zro)
    out = _post(agg2, cnt, h, Wroot2, b2.reshape(1, D), x_p)
    return out[:N]
